# baseline (speedup 1.0000x reference)
import functools

import jax
import jax.numpy as jnp
from jax import lax
from jax.experimental import pallas as pl
from jax.experimental.pallas import tpu as pltpu
from jax.experimental.pallas import tpu_sc as plsc

NC = 2
NS = 16
NW = NC * NS
G = 16


@functools.lru_cache(maxsize=None)
def _make_gather(V, D, B):
    assert B % (NW * G) == 0
    b_per_w = B // NW
    n_groups = b_per_w // G

    mesh = plsc.VectorSubcoreMesh(core_axis_name="c", subcore_axis_name="s")

    @functools.partial(
        pl.kernel,
        out_type=jax.ShapeDtypeStruct((B, D), jnp.float32),
        mesh=mesh,
        compiler_params=pltpu.CompilerParams(use_tc_tiling_on_sc=False),
        scratch_types=[
            pltpu.VMEM((b_per_w,), jnp.int32),
            pltpu.SemaphoreType.DMA,
        ],
    )
    def gather_kernel(idx_hbm, table_hbm, out_hbm, idx_v, sem):
        wid = lax.axis_index("s") * NC + lax.axis_index("c")
        base = wid * b_per_w

        pltpu.sync_copy(idx_hbm.at[pl.ds(base, b_per_w)], idx_v)

        def fire_group(g):
            off = pl.multiple_of(g * G, G)
            v = idx_v[pl.ds(off, G)]
            for j in range(G):
                pltpu.async_copy(
                    table_hbm.at[pl.ds(v[j], 1)],
                    out_hbm.at[pl.ds(base + off + j, 1)],
                    sem,
                )

        def drain_group():
            pltpu.make_async_copy(
                table_hbm.at[pl.ds(0, G)], out_hbm.at[pl.ds(base, G)], sem
            ).wait()

        fire_group(0)

        @pl.loop(1, n_groups)
        def _(g):
            fire_group(g)
            drain_group()

        drain_group()

    return gather_kernel


def kernel(indices, table):
    BATCH, SEQ = indices.shape
    V, D = table.shape
    B = BATCH * SEQ
    idx_flat = indices.reshape(B).astype(jnp.int32)
    out = _make_gather(V, D, B)(idx_flat, table)
    return out.reshape(BATCH, SEQ, D)


# speedup vs baseline: 1.0007x; 1.0007x over previous
import functools

import jax
import jax.numpy as jnp
from jax import lax
from jax.experimental import pallas as pl
from jax.experimental.pallas import tpu as pltpu
from jax.experimental.pallas import tpu_sc as plsc

NC = 2
NS = 16
NW = NC * NS
G = 16


@functools.lru_cache(maxsize=None)
def _make_gather(V, D, B):
    assert B % (NW * G) == 0
    b_per_w = B // NW
    n_groups = b_per_w // G

    mesh = plsc.VectorSubcoreMesh(core_axis_name="c", subcore_axis_name="s")

    @functools.partial(
        pl.kernel,
        out_type=jax.ShapeDtypeStruct((B, D), jnp.float32),
        mesh=mesh,
        compiler_params=pltpu.CompilerParams(use_tc_tiling_on_sc=False),
        scratch_types=[
            pltpu.VMEM((b_per_w,), jnp.int32),
            pltpu.SemaphoreType.DMA,
        ],
    )
    def gather_kernel(idx_hbm, table_hbm, out_hbm, idx_v, sem):
        wid = lax.axis_index("s") * NC + lax.axis_index("c")
        base = wid * b_per_w

        pltpu.sync_copy(idx_hbm.at[pl.ds(base, b_per_w)], idx_v)

        def fire_group(g):
            off = pl.multiple_of(g * G, G)
            v = idx_v[pl.ds(off, G)]
            for j in range(G):
                pltpu.async_copy(
                    table_hbm.at[pl.ds(v[j], 1)],
                    out_hbm.at[pl.ds(base + off + j, 1)],
                    sem,
                )

        def drain_group():
            pltpu.make_async_copy(
                table_hbm.at[pl.ds(0, G)], out_hbm.at[pl.ds(base, G)], sem
            ).wait()

        LAG = 16
        for g in range(LAG):
            fire_group(g)

        @pl.loop(LAG, n_groups)
        def _(g):
            fire_group(g)
            drain_group()

        for _ in range(LAG):
            drain_group()

    return gather_kernel


def kernel(indices, table):
    BATCH, SEQ = indices.shape
    V, D = table.shape
    B = BATCH * SEQ
    idx_flat = indices.reshape(B).astype(jnp.int32)
    out = _make_gather(V, D, B)(idx_flat, table)
    return out.reshape(BATCH, SEQ, D)


# speedup vs baseline: 5.5705x; 5.5664x over previous
import functools

import jax
import jax.numpy as jnp
from jax import lax
from jax.experimental import pallas as pl
from jax.experimental.pallas import tpu as pltpu
from jax.experimental.pallas import tpu_sc as plsc

NC = 2
NS = 16
NW = NC * NS
L = 16
R = 64
GPR = 20


@functools.lru_cache(maxsize=None)
def _make_gather(V, D, B):
    assert D == 300 and (V * D) % L == 0
    assert B % (NW * R) == 0
    b_per_w = B // NW
    n_chunks = b_per_w // R
    assert n_chunks % 2 == 0
    NIDX = R * GPR
    NDW = R * D
    n_full = D // L
    tail = D - n_full * L

    mesh = plsc.VectorSubcoreMesh(core_axis_name="c", subcore_axis_name="s")

    @functools.partial(
        pl.kernel,
        out_type=jax.ShapeDtypeStruct((B * D,), jnp.float32),
        mesh=mesh,
        compiler_params=pltpu.CompilerParams(
            use_tc_tiling_on_sc=False, needs_layout_passes=False
        ),
        scratch_types=[
            pltpu.VMEM((b_per_w,), jnp.int32),
            [pltpu.VMEM((NIDX,), jnp.int32) for _ in range(2)],
            [pltpu.VMEM((NIDX, L), jnp.float32) for _ in range(2)],
            [pltpu.VMEM((NDW,), jnp.float32) for _ in range(2)],
            [pltpu.SemaphoreType.DMA for _ in range(2)],
            [pltpu.SemaphoreType.DMA for _ in range(2)],
        ],
    )
    def gather_kernel(idx_hbm, tabg_hbm, out_hbm, idx_v, idxg, win, dense,
                      gsem, osem):
        wid = lax.axis_index("s") * NC + lax.axis_index("c")
        base = wid * b_per_w

        pltpu.sync_copy(idx_hbm.at[pl.ds(base, b_per_w)], idx_v)

        lane = lax.iota(jnp.int32, L)
        dst20 = lane * GPR
        tailmask = lane < tail

        def load_group(c, g):
            off = pl.multiple_of(c * R + g * L, L)
            return idx_v[pl.ds(off, L)]

        def build_and_fire(c, b):
            for g in range(R // L):
                iv = load_group(c, g)
                g0 = (iv * 75) >> 2
                dbase = dst20 + (GPR * L * g)
                for t in range(GPR):
                    plsc.store_scatter(idxg[b], [dbase + t], g0 + t)
            pltpu.async_copy(tabg_hbm.at[idxg[b]], win[b], gsem[b])

        def gather_wait(b):
            pltpu.make_async_copy(tabg_hbm.at[idxg[b]], win[b], gsem[b]).wait()

        def out_ref(c):
            off = pl.multiple_of((base + c * R) * D, 8)
            return out_hbm.at[pl.ds(off, NDW)]

        def out_fire(c, b):
            pltpu.async_copy(dense[b], out_ref(c), osem[b])

        def out_wait(c, b):
            pltpu.make_async_copy(dense[b], out_ref(c), osem[b]).wait()

        def realign(c, b):
            for j in range(L):
                @pl.loop(0, R // L)
                def _(g):
                    iv = load_group(c, g)
                    s = ((iv * 12) & 15)[j]
                    qsrc = s + lane
                    qrow0 = (qsrc >> 4) + (g * (L * GPR) + j * GPR)
                    qlan = qsrc & 15
                    dst0 = g * (L * D) + (j * D) + lane

                    def step(i, carry):
                        qr, dv = carry
                        val = plsc.load_gather(win[b], [qr, qlan])
                        plsc.store_scatter(dense[b], [dv], val)
                        return (qr + 1, dv + L)

                    qr, dv = pl.loop(
                        0, n_full, init_carry=(qrow0, dst0), unroll=6
                    )(step)
                    val = plsc.load_gather(win[b], [qr, qlan])
                    plsc.store_scatter(dense[b], [dv], val, mask=tailmask)

        build_and_fire(0, 0)
        build_and_fire(1, 1)

        @pl.loop(0, n_chunks // 2)
        def _(q):
            for b in range(2):
                c = 2 * q + b
                gather_wait(b)

                @pl.when(c >= 2)
                def _():
                    out_wait(c - 2, b)

                realign(c, b)
                out_fire(c, b)

                @pl.when(c + 2 < n_chunks)
                def _():
                    build_and_fire(c + 2, b)

        out_wait(n_chunks - 2, 0)
        out_wait(n_chunks - 1, 1)

    return gather_kernel


def kernel(indices, table):
    BATCH, SEQ = indices.shape
    V, D = table.shape
    B = BATCH * SEQ
    idx_flat = indices.reshape(B).astype(jnp.int32)
    tabg = table.reshape(V * D // L, L)
    out = _make_gather(V, D, B)(idx_flat, tabg)
    return out.reshape(BATCH, SEQ, D)
